# Initial kernel scaffold; baseline (speedup 1.0000x reference)
import jax
import jax.numpy as jnp
from jax.experimental import pallas as pl


def kernel(features, subgraph_adj, subgraph_norm, node_subgraph, node_list, edge_index, perm, W_gcn, b_gcn):
    raise NotImplementedError("write your pallas kernel here")


# speedup vs baseline: 3.1937x; 3.1937x over previous
import jax
import jax.numpy as jnp
from jax.experimental import pallas as pl
from jax.experimental.pallas import tpu as pltpu

N = 10000
S = 100
H = 128


def _loss_body(pos_ref, neg_ref, sum_ref, g_ref, ploss_ref, nloss_ref):
    pos = pos_ref[...]
    neg = neg_ref[...]
    ge = sum_ref[...]
    g = g_ref[...]
    P = jax.lax.dot_general(pos, ge, (((1,), (1,)), ((), ())),
                            preferred_element_type=jnp.float32)
    Q = jax.lax.dot_general(neg, ge, (((1,), (1,)), ((), ())),
                            preferred_element_type=jnp.float32)
    mask = (jax.lax.broadcasted_iota(jnp.int32, P.shape, 1) == g).astype(jnp.float32)
    sp = jnp.sum(P * mask, axis=1, keepdims=True)
    sn = jnp.sum(Q * mask, axis=1, keepdims=True)

    def softplus(x):
        return jnp.maximum(x, 0.0) + jnp.log1p(jnp.exp(-jnp.abs(x)))

    ploss_ref[...] = softplus(-sp)
    nloss_ref[...] = softplus(sn)


def _losses(positive, negative, graph_embeds, node_subgraph):
    blk = 1000
    grid = (N // blk,)
    ploss, nloss = pl.pallas_call(
        _loss_body,
        grid=grid,
        in_specs=[
            pl.BlockSpec((blk, H), lambda i: (i, 0)),
            pl.BlockSpec((blk, H), lambda i: (i, 0)),
            pl.BlockSpec((S, H), lambda i: (0, 0)),
            pl.BlockSpec((blk, 1), lambda i: (i, 0)),
        ],
        out_specs=[
            pl.BlockSpec((blk, 1), lambda i: (i, 0)),
            pl.BlockSpec((blk, 1), lambda i: (i, 0)),
        ],
        out_shape=[
            jax.ShapeDtypeStruct((N, 1), jnp.float32),
            jax.ShapeDtypeStruct((N, 1), jnp.float32),
        ],
    )(positive, negative, graph_embeds, node_subgraph.reshape(N, 1))
    return ploss[:, 0], nloss[:, 0]


def kernel(features, subgraph_adj, subgraph_norm, node_subgraph, node_list,
           edge_index, perm, W_gcn, b_gcn):
    src = edge_index[0]
    dst = edge_index[1]
    deg = jnp.zeros((N,), dtype=jnp.float32).at[dst].add(1.0)
    deg = jnp.clip(deg, 1.0)
    isd = 1.0 / jnp.sqrt(deg)

    G = features @ W_gcn
    T_pos = (G + b_gcn) * isd[:, None]
    T_neg = (G[perm] + b_gcn) * isd[:, None]

    msg = jnp.concatenate([T_pos, T_neg], axis=1)[src]
    out = jax.ops.segment_sum(msg, dst, num_segments=N)
    positive = jax.nn.relu(out[:, :H] * isd[:, None])
    negative = jax.nn.relu(out[:, H:] * isd[:, None])

    graph_embeds = (subgraph_adj @ positive) / subgraph_norm
    graph_embeds = jax.nn.relu(graph_embeds)

    ploss, nloss = _losses(positive, negative, graph_embeds, node_subgraph)
    return (ploss, nloss)


# speedup vs baseline: 9.6202x; 3.0122x over previous
import jax
import jax.numpy as jnp
from jax.experimental import pallas as pl
from jax.experimental.pallas import tpu as pltpu
from jax.experimental.pallas import tpu_sc as plsc

N = 10000
E = 320000
H = 128
S = 100

NC = 2
NS = 16
CH = 128
E_PAD = 323584
N_PAD = 10240
RPT = N_PAD // NS

def _sc_mesh():
    return plsc.VectorSubcoreMesh(core_axis_name="c", subcore_axis_name="s",
                                  num_cores=NC, num_subcores=NS)


def _deg_body(dst4_h, zeros_h, ones_h, deg_h, dst_v, ones_v, deg_sp):
    c = jax.lax.axis_index("c")
    s = jax.lax.axis_index("s")
    wid = c * NS + s
    pltpu.sync_copy(ones_h, ones_v)
    pltpu.sync_copy(zeros_h.at[pl.ds(s * RPT, RPT)], deg_sp.at[pl.ds(s * RPT, RPT)])
    plsc.subcore_barrier()

    def body(i, carry):
        pltpu.sync_copy(dst4_h.at[wid].at[i], dst_v)
        pltpu.sync_copy(ones_v, deg_sp.at[dst_v], add=True)
        return carry

    jax.lax.fori_loop(0, E_PAD // (NC * NS * CH), body, 0)
    plsc.subcore_barrier()
    pltpu.sync_copy(deg_sp.at[pl.ds(s * RPT, RPT)],
                    deg_h.at[c].at[pl.ds(s * RPT, RPT)])


def _permgather_body(g_h, perm_h, gs_h, idx_v, buf_v, sem):
    c = jax.lax.axis_index("c")
    s = jax.lax.axis_index("s")

    @pl.when(c == 0)
    def _():
        base = s * 624
        pltpu.sync_copy(perm_h.at[pl.ds(base, 640)], idx_v)

        def body(k, carry):
            pltpu.async_copy(g_h.at[idx_v.at[pl.ds(k * CH, CH)]], buf_v, sem).wait()
            pltpu.sync_copy(buf_v, gs_h.at[pl.ds(base + k * CH, CH)])
            return carry

        jax.lax.fori_loop(0, 5, body, 0)


def _edge_body(t_h, src3_h, dst3_h, zeros_h, s_h, src_v, dst_v, buf_v, sem, acc):
    c = jax.lax.axis_index("c")
    s = jax.lax.axis_index("s")
    pltpu.sync_copy(zeros_h.at[pl.ds(s * RPT, RPT)], acc.at[pl.ds(s * RPT, RPT)])
    plsc.subcore_barrier()

    def body(i, carry):
        pltpu.sync_copy(src3_h.at[s].at[i], src_v)
        pltpu.sync_copy(dst3_h.at[s].at[i], dst_v)
        pltpu.async_copy(t_h.at[c].at[src_v], buf_v, sem).wait()
        pltpu.sync_copy(buf_v, acc.at[dst_v], add=True)
        return carry

    jax.lax.fori_loop(0, E_PAD // (NS * CH), body, 0)
    plsc.subcore_barrier()
    pltpu.sync_copy(acc.at[pl.ds(s * RPT, RPT)],
                    s_h.at[c].at[pl.ds(s * RPT, RPT)])


def _matmul_body(x_ref, w_ref, o_ref):
    o_ref[...] = jnp.dot(x_ref[...], w_ref[...], preferred_element_type=jnp.float32)


def _table_body(g_ref, gs_ref, deg_ref, b_ref, t_ref, isd_ref):
    deg = deg_ref[0, :, :1] + deg_ref[1, :, :1]
    isd = jax.lax.rsqrt(jnp.maximum(deg, 1.0))
    b = b_ref[...]
    t_ref[0] = (g_ref[...] + b) * isd
    t_ref[1] = (gs_ref[...] + b) * isd
    isd_ref[...] = isd


def _embed_body(s_ref, isd_ref, adj_ref, norm_ref, pos_ref, neg_ref, ge_ref, acc):
    i = pl.program_id(0)
    blk = s_ref.shape[1]
    isd = isd_ref[...]
    rows = jax.lax.broadcasted_iota(jnp.int32, (blk, 1), 0) + i * blk
    valid = rows < N
    pos = jnp.where(valid, jnp.maximum(s_ref[0] * isd, 0.0), 0.0)
    neg = jnp.where(valid, jnp.maximum(s_ref[1] * isd, 0.0), 0.0)
    pos_ref[...] = pos
    neg_ref[...] = neg
    part = jnp.dot(adj_ref[...], pos, preferred_element_type=jnp.float32)

    @pl.when(i == 0)
    def _():
        acc[...] = part

    @pl.when(i > 0)
    def _():
        acc[...] += part

    @pl.when(i == pl.num_programs(0) - 1)
    def _():
        ge_ref[...] = jnp.maximum(acc[...] / norm_ref[...], 0.0)


def _loss_body(pos_ref, neg_ref, ge_ref, g_ref, ploss_ref, nloss_ref):
    ge = ge_ref[...]
    g = g_ref[...]
    P = jax.lax.dot_general(pos_ref[...], ge, (((1,), (1,)), ((), ())),
                            preferred_element_type=jnp.float32)
    Q = jax.lax.dot_general(neg_ref[...], ge, (((1,), (1,)), ((), ())),
                            preferred_element_type=jnp.float32)
    mask = (jax.lax.broadcasted_iota(jnp.int32, P.shape, 1) == g).astype(jnp.float32)
    sp = jnp.sum(P * mask, axis=1, keepdims=True)
    sn = jnp.sum(Q * mask, axis=1, keepdims=True)

    def softplus(x):
        return jnp.maximum(x, 0.0) + jnp.log1p(jnp.exp(-jnp.abs(x)))

    ploss_ref[...] = softplus(-sp)
    nloss_ref[...] = softplus(sn)


def kernel(features, subgraph_adj, subgraph_norm, node_subgraph, node_list,
           edge_index, perm, W_gcn, b_gcn):
    pad = E_PAD - E
    src_p = jnp.concatenate([edge_index[0], jnp.zeros((pad,), jnp.int32)])
    dst_p = jnp.concatenate([edge_index[1], jnp.full((pad,), N, jnp.int32)])
    dst4 = dst_p.reshape(NC * NS, E_PAD // (NC * NS * CH), CH)
    src3 = src_p.reshape(NS, E_PAD // (NS * CH), CH)
    dst3 = dst_p.reshape(NS, E_PAD // (NS * CH), CH)
    z16 = jnp.zeros((N_PAD, 16), jnp.float32)
    z128 = jnp.zeros((N_PAD, H), jnp.float32)
    ones16 = jnp.ones((CH, H), jnp.float32)

    degA = pl.kernel(
        _deg_body,
        out_type=jax.ShapeDtypeStruct((NC, N_PAD, H), jnp.float32),
        mesh=_sc_mesh(),
        scratch_types=[
            pltpu.VMEM((CH,), jnp.int32),
            pltpu.VMEM((CH, H), jnp.float32),
            pltpu.VMEM_SHARED((N_PAD, H), jnp.float32),
        ],
    )(dst4, z128, ones16)

    G = pl.pallas_call(
        _matmul_body,
        grid=(10,),
        in_specs=[pl.BlockSpec((N // 10, H), lambda i: (i, 0)),
                  pl.BlockSpec((H, H), lambda i: (0, 0))],
        out_specs=pl.BlockSpec((N // 10, H), lambda i: (i, 0)),
        out_shape=jax.ShapeDtypeStruct((N, H), jnp.float32),
    )(features, W_gcn)

    Gs = pl.kernel(
        _permgather_body,
        out_type=jax.ShapeDtypeStruct((N, H), jnp.float32),
        mesh=_sc_mesh(),
        scratch_types=[
            pltpu.VMEM((640,), jnp.int32),
            pltpu.VMEM((CH, H), jnp.float32),
            pltpu.SemaphoreType.DMA,
        ],
    )(G, perm)

    T, isd = pl.pallas_call(
        _table_body,
        grid=(10,),
        in_specs=[
            pl.BlockSpec((N // 10, H), lambda i: (i, 0)),
            pl.BlockSpec((N // 10, H), lambda i: (i, 0)),
            pl.BlockSpec((NC, N // 10, H), lambda i: (0, i, 0)),
            pl.BlockSpec((1, H), lambda i: (0, 0)),
        ],
        out_specs=[
            pl.BlockSpec((NC, N // 10, H), lambda i: (0, i, 0)),
            pl.BlockSpec((N // 10, 1), lambda i: (i, 0)),
        ],
        out_shape=[
            jax.ShapeDtypeStruct((NC, N, H), jnp.float32),
            jax.ShapeDtypeStruct((N, 1), jnp.float32),
        ],
    )(G, Gs, degA, b_gcn.reshape(1, H))

    S_out = pl.kernel(
        _edge_body,
        out_type=jax.ShapeDtypeStruct((NC, N_PAD, H), jnp.float32),
        mesh=_sc_mesh(),
        scratch_types=[
            pltpu.VMEM((CH,), jnp.int32),
            pltpu.VMEM((CH,), jnp.int32),
            pltpu.VMEM((CH, H), jnp.float32),
            pltpu.SemaphoreType.DMA,
            pltpu.VMEM_SHARED((N_PAD, H), jnp.float32),
        ],
    )(T, src3, dst3, z128)

    adj_p = jnp.pad(subgraph_adj, ((0, 0), (0, N_PAD - N)))
    fb = N_PAD // 8
    pos, neg, ge = pl.pallas_call(
        _embed_body,
        grid=(8,),
        in_specs=[
            pl.BlockSpec((NC, fb, H), lambda i: (0, i, 0)),
            pl.BlockSpec((fb, 1), lambda i: (i, 0)),
            pl.BlockSpec((S, fb), lambda i: (0, i)),
            pl.BlockSpec((S, 1), lambda i: (0, 0)),
        ],
        out_specs=[
            pl.BlockSpec((fb, H), lambda i: (i, 0)),
            pl.BlockSpec((fb, H), lambda i: (i, 0)),
            pl.BlockSpec((S, H), lambda i: (0, 0)),
        ],
        out_shape=[
            jax.ShapeDtypeStruct((N_PAD, H), jnp.float32),
            jax.ShapeDtypeStruct((N_PAD, H), jnp.float32),
            jax.ShapeDtypeStruct((S, H), jnp.float32),
        ],
        scratch_shapes=[pltpu.VMEM((S, H), jnp.float32)],
    )(S_out, isd, adj_p, subgraph_norm)

    ploss, nloss = pl.pallas_call(
        _loss_body,
        grid=(10,),
        in_specs=[
            pl.BlockSpec((N // 10, H), lambda i: (i, 0)),
            pl.BlockSpec((N // 10, H), lambda i: (i, 0)),
            pl.BlockSpec((S, H), lambda i: (0, 0)),
            pl.BlockSpec((N // 10, 1), lambda i: (i, 0)),
        ],
        out_specs=[
            pl.BlockSpec((N // 10, 1), lambda i: (i, 0)),
            pl.BlockSpec((N // 10, 1), lambda i: (i, 0)),
        ],
        out_shape=[
            jax.ShapeDtypeStruct((N, 1), jnp.float32),
            jax.ShapeDtypeStruct((N, 1), jnp.float32),
        ],
    )(pos, neg, ge, node_subgraph.reshape(N, 1))

    return (ploss[:, 0], nloss[:, 0])


# speedup vs baseline: 12.4622x; 1.2954x over previous
import jax
import jax.numpy as jnp
from jax.experimental import pallas as pl
from jax.experimental.pallas import tpu as pltpu
from jax.experimental.pallas import tpu_sc as plsc

N = 10000
E = 320000
H = 128
S = 100

NC = 2
NS = 16
CH = 128
E_PAD = 323584
N_PAD = 10240
RPT = N_PAD // NS

def _sc_mesh():
    return plsc.VectorSubcoreMesh(core_axis_name="c", subcore_axis_name="s",
                                  num_cores=NC, num_subcores=NS)


def _deg_body(dst4_h, zeros_h, ones_h, deg_h, dst_v, ones_v, deg_sp):
    c = jax.lax.axis_index("c")
    s = jax.lax.axis_index("s")
    wid = c * NS + s
    pltpu.sync_copy(ones_h, ones_v)
    pltpu.sync_copy(zeros_h.at[pl.ds(s * RPT, RPT)], deg_sp.at[pl.ds(s * RPT, RPT)])
    plsc.subcore_barrier()

    def body(i, carry):
        pltpu.sync_copy(dst4_h.at[wid].at[i], dst_v)
        pltpu.sync_copy(ones_v, deg_sp.at[dst_v], add=True)
        return carry

    jax.lax.fori_loop(0, E_PAD // (NC * NS * CH), body, 0)
    plsc.subcore_barrier()
    pltpu.sync_copy(deg_sp.at[pl.ds(s * RPT, RPT)],
                    deg_h.at[c].at[pl.ds(s * RPT, RPT)])


def _permgather_body(g_h, perm_h, gs_h, idx_v, buf_v, sem):
    c = jax.lax.axis_index("c")
    s = jax.lax.axis_index("s")

    @pl.when(c == 0)
    def _():
        base = s * 624
        pltpu.sync_copy(perm_h.at[pl.ds(base, 640)], idx_v)

        def body(k, carry):
            pltpu.async_copy(g_h.at[idx_v.at[pl.ds(k * CH, CH)]], buf_v, sem).wait()
            pltpu.sync_copy(buf_v, gs_h.at[pl.ds(base + k * CH, CH)])
            return carry

        jax.lax.fori_loop(0, 5, body, 0)


def _edge_body(t_h, src3_h, dst3_h, zeros_h, s_h,
               src_a, dst_a, buf_a, sem_a, src_b, dst_b, buf_b, sem_b, acc):
    c = jax.lax.axis_index("c")
    s = jax.lax.axis_index("s")
    pltpu.sync_copy(zeros_h.at[pl.ds(s * RPT, RPT)], acc.at[pl.ds(s * RPT, RPT)])
    plsc.subcore_barrier()

    npairs = E_PAD // (NS * CH) // 2

    pltpu.sync_copy(src3_h.at[s].at[0], src_a)
    pltpu.sync_copy(dst3_h.at[s].at[0], dst_a)
    pltpu.async_copy(t_h.at[c].at[src_a], buf_a, sem_a)

    def body(j, carry):
        i = 2 * j
        pltpu.sync_copy(src3_h.at[s].at[i + 1], src_b)
        pltpu.sync_copy(dst3_h.at[s].at[i + 1], dst_b)
        pltpu.async_copy(t_h.at[c].at[src_b], buf_b, sem_b)
        pltpu.make_async_copy(t_h.at[c].at[src_a], buf_a, sem_a).wait()
        pltpu.sync_copy(buf_a, acc.at[dst_a], add=True)

        @pl.when(j < npairs - 1)
        def _():
            pltpu.sync_copy(src3_h.at[s].at[i + 2], src_a)
            pltpu.sync_copy(dst3_h.at[s].at[i + 2], dst_a)
            pltpu.async_copy(t_h.at[c].at[src_a], buf_a, sem_a)

        pltpu.make_async_copy(t_h.at[c].at[src_b], buf_b, sem_b).wait()
        pltpu.sync_copy(buf_b, acc.at[dst_b], add=True)
        return carry

    jax.lax.fori_loop(0, npairs, body, 0)
    plsc.subcore_barrier()
    pltpu.sync_copy(acc.at[pl.ds(s * RPT, RPT)],
                    s_h.at[c].at[pl.ds(s * RPT, RPT)])


def _matmul_body(x_ref, w_ref, o_ref):
    o_ref[...] = jnp.dot(x_ref[...], w_ref[...], preferred_element_type=jnp.float32)


def _table_body(g_ref, gs_ref, deg_ref, b_ref, t_ref, isd_ref):
    deg = deg_ref[0, :, :1] + deg_ref[1, :, :1]
    isd = jax.lax.rsqrt(jnp.maximum(deg, 1.0))
    b = b_ref[...]
    t_ref[0] = (g_ref[...] + b) * isd
    t_ref[1] = (gs_ref[...] + b) * isd
    isd_ref[...] = isd


def _embed_body(s_ref, isd_ref, adj_ref, norm_ref, pos_ref, neg_ref, ge_ref, acc):
    i = pl.program_id(0)
    blk = s_ref.shape[1]
    isd = isd_ref[...]
    rows = jax.lax.broadcasted_iota(jnp.int32, (blk, 1), 0) + i * blk
    valid = rows < N
    pos = jnp.where(valid, jnp.maximum(s_ref[0] * isd, 0.0), 0.0)
    neg = jnp.where(valid, jnp.maximum(s_ref[1] * isd, 0.0), 0.0)
    pos_ref[...] = pos
    neg_ref[...] = neg
    part = jnp.dot(adj_ref[...], pos, preferred_element_type=jnp.float32)

    @pl.when(i == 0)
    def _():
        acc[...] = part

    @pl.when(i > 0)
    def _():
        acc[...] += part

    @pl.when(i == pl.num_programs(0) - 1)
    def _():
        ge_ref[...] = jnp.maximum(acc[...] / norm_ref[...], 0.0)


def _loss_body(pos_ref, neg_ref, ge_ref, g_ref, ploss_ref, nloss_ref):
    ge = ge_ref[...]
    g = g_ref[...]
    P = jax.lax.dot_general(pos_ref[...], ge, (((1,), (1,)), ((), ())),
                            preferred_element_type=jnp.float32)
    Q = jax.lax.dot_general(neg_ref[...], ge, (((1,), (1,)), ((), ())),
                            preferred_element_type=jnp.float32)
    mask = (jax.lax.broadcasted_iota(jnp.int32, P.shape, 1) == g).astype(jnp.float32)
    sp = jnp.sum(P * mask, axis=1, keepdims=True)
    sn = jnp.sum(Q * mask, axis=1, keepdims=True)

    def softplus(x):
        return jnp.maximum(x, 0.0) + jnp.log1p(jnp.exp(-jnp.abs(x)))

    ploss_ref[...] = softplus(-sp)
    nloss_ref[...] = softplus(sn)


def kernel(features, subgraph_adj, subgraph_norm, node_subgraph, node_list,
           edge_index, perm, W_gcn, b_gcn):
    pad = E_PAD - E
    src_p = jnp.concatenate([edge_index[0], jnp.zeros((pad,), jnp.int32)])
    dst_p = jnp.concatenate([edge_index[1], jnp.full((pad,), N, jnp.int32)])
    dst4 = dst_p.reshape(NC * NS, E_PAD // (NC * NS * CH), CH)
    src3 = src_p.reshape(NS, E_PAD // (NS * CH), CH)
    dst3 = dst_p.reshape(NS, E_PAD // (NS * CH), CH)
    z16 = jnp.zeros((N_PAD, 16), jnp.float32)
    z128 = jnp.zeros((N_PAD, H), jnp.float32)
    ones16 = jnp.ones((CH, H), jnp.float32)

    degA = pl.kernel(
        _deg_body,
        out_type=jax.ShapeDtypeStruct((NC, N_PAD, H), jnp.float32),
        mesh=_sc_mesh(),
        scratch_types=[
            pltpu.VMEM((CH,), jnp.int32),
            pltpu.VMEM((CH, H), jnp.float32),
            pltpu.VMEM_SHARED((N_PAD, H), jnp.float32),
        ],
    )(dst4, z128, ones16)

    G = pl.pallas_call(
        _matmul_body,
        grid=(10,),
        in_specs=[pl.BlockSpec((N // 10, H), lambda i: (i, 0)),
                  pl.BlockSpec((H, H), lambda i: (0, 0))],
        out_specs=pl.BlockSpec((N // 10, H), lambda i: (i, 0)),
        out_shape=jax.ShapeDtypeStruct((N, H), jnp.float32),
    )(features, W_gcn)

    Gs = pl.kernel(
        _permgather_body,
        out_type=jax.ShapeDtypeStruct((N, H), jnp.float32),
        mesh=_sc_mesh(),
        scratch_types=[
            pltpu.VMEM((640,), jnp.int32),
            pltpu.VMEM((CH, H), jnp.float32),
            pltpu.SemaphoreType.DMA,
        ],
    )(G, perm)

    T, isd = pl.pallas_call(
        _table_body,
        grid=(10,),
        in_specs=[
            pl.BlockSpec((N // 10, H), lambda i: (i, 0)),
            pl.BlockSpec((N // 10, H), lambda i: (i, 0)),
            pl.BlockSpec((NC, N // 10, H), lambda i: (0, i, 0)),
            pl.BlockSpec((1, H), lambda i: (0, 0)),
        ],
        out_specs=[
            pl.BlockSpec((NC, N // 10, H), lambda i: (0, i, 0)),
            pl.BlockSpec((N // 10, 1), lambda i: (i, 0)),
        ],
        out_shape=[
            jax.ShapeDtypeStruct((NC, N, H), jnp.float32),
            jax.ShapeDtypeStruct((N, 1), jnp.float32),
        ],
    )(G, Gs, degA, b_gcn.reshape(1, H))

    S_out = pl.kernel(
        _edge_body,
        out_type=jax.ShapeDtypeStruct((NC, N_PAD, H), jnp.float32),
        mesh=_sc_mesh(),
        scratch_types=[
            pltpu.VMEM((CH,), jnp.int32),
            pltpu.VMEM((CH,), jnp.int32),
            pltpu.VMEM((CH, H), jnp.float32),
            pltpu.SemaphoreType.DMA,
            pltpu.VMEM((CH,), jnp.int32),
            pltpu.VMEM((CH,), jnp.int32),
            pltpu.VMEM((CH, H), jnp.float32),
            pltpu.SemaphoreType.DMA,
            pltpu.VMEM_SHARED((N_PAD, H), jnp.float32),
        ],
    )(T, src3, dst3, z128)

    adj_p = jnp.pad(subgraph_adj, ((0, 0), (0, N_PAD - N)))
    fb = N_PAD // 8
    pos, neg, ge = pl.pallas_call(
        _embed_body,
        grid=(8,),
        in_specs=[
            pl.BlockSpec((NC, fb, H), lambda i: (0, i, 0)),
            pl.BlockSpec((fb, 1), lambda i: (i, 0)),
            pl.BlockSpec((S, fb), lambda i: (0, i)),
            pl.BlockSpec((S, 1), lambda i: (0, 0)),
        ],
        out_specs=[
            pl.BlockSpec((fb, H), lambda i: (i, 0)),
            pl.BlockSpec((fb, H), lambda i: (i, 0)),
            pl.BlockSpec((S, H), lambda i: (0, 0)),
        ],
        out_shape=[
            jax.ShapeDtypeStruct((N_PAD, H), jnp.float32),
            jax.ShapeDtypeStruct((N_PAD, H), jnp.float32),
            jax.ShapeDtypeStruct((S, H), jnp.float32),
        ],
        scratch_shapes=[pltpu.VMEM((S, H), jnp.float32)],
    )(S_out, isd, adj_p, subgraph_norm)

    ploss, nloss = pl.pallas_call(
        _loss_body,
        grid=(10,),
        in_specs=[
            pl.BlockSpec((N // 10, H), lambda i: (i, 0)),
            pl.BlockSpec((N // 10, H), lambda i: (i, 0)),
            pl.BlockSpec((S, H), lambda i: (0, 0)),
            pl.BlockSpec((N // 10, 1), lambda i: (i, 0)),
        ],
        out_specs=[
            pl.BlockSpec((N // 10, 1), lambda i: (i, 0)),
            pl.BlockSpec((N // 10, 1), lambda i: (i, 0)),
        ],
        out_shape=[
            jax.ShapeDtypeStruct((N, 1), jnp.float32),
            jax.ShapeDtypeStruct((N, 1), jnp.float32),
        ],
    )(pos, neg, ge, node_subgraph.reshape(N, 1))

    return (ploss[:, 0], nloss[:, 0])
